# baseline (speedup 1.0000x reference)
import functools

import jax
import jax.numpy as jnp
from jax import lax
from jax.experimental import pallas as pl
from jax.experimental.pallas import tpu as pltpu
from jax.experimental.pallas import tpu_sc as plsc

USER_NUM = 25000
ITEM_NUM = 25000
N_NODES = USER_NUM + ITEM_NUM
N_EDGES = 800000
EMB = 64
HALF = 32
N_LAYERS = 3

NC = 2
NS = 16
EBLK = 128
NBLK = 393
NBLK_IO = NBLK + 1
E_PAD = NS * NBLK * EBLK
N_PAD = 50176
ROWS_PER_TILE = N_PAD // NS
UROWS = 1568
ULAST = USER_NUM - 15 * UROWS
UCHUNK = 112
LCHUNK = USER_NUM - (15 * UROWS + 13 * UCHUNK)


def _body(ego0, srcs, dsts, vals, zrows, out, uout, iout,
          acc, sb0, sb1, sb2, db0, db1, db2, vb0, vb1, vb2,
          dstb0, dstb1, dstb2, rows, mba, mbb,
          si0, si1, si2, sj0, sj1, sj2, sk0, sk1, sk2,
          sg0, sg1, sg2, ss0, ss1, ss2):
    c = lax.axis_index("c")
    s = lax.axis_index("s")
    sbuf = (sb0, sb1, sb2)
    dbuf = (db0, db1, db2)
    vbuf = (vb0, vb1, vb2)
    dstb = (dstb0, dstb1, dstb2)
    sem_i = (si0, si1, si2)
    sem_j = (sj0, sj1, sj2)
    sem_k = (sk0, sk1, sk2)
    sem_g = (sg0, sg1, sg2)
    sem_s = (ss0, ss1, ss2)

    def zero_acc():
        pltpu.sync_copy(zrows, acc.at[pl.ds(s * ROWS_PER_TILE,
                                            ROWS_PER_TILE)])

    def item_copy(src_table, l):
        @pl.when(s < NS - 1)
        def _():
            pltpu.sync_copy(
                src_table.at[pl.ds(USER_NUM + s * UROWS, UROWS)],
                iout.at[l].at[pl.ds(s * UROWS, UROWS),
                              pl.ds(HALF * c, HALF)])

        @pl.when(s == NS - 1)
        def _():
            pltpu.sync_copy(
                src_table.at[pl.ds(USER_NUM + 15 * UROWS, ULAST)],
                iout.at[l].at[pl.ds(15 * UROWS, ULAST),
                              pl.ds(HALF * c, HALF)])

    zero_acc()
    item_copy(ego0.at[c], 0)
    plsc.subcore_barrier()

    class idx_copy:
        def __init__(self, j, m):
            g = s * NBLK_IO + j
            self.a = pltpu.make_async_copy(srcs.at[g], sbuf[m], sem_i[m])
            self.b = pltpu.make_async_copy(dsts.at[g], dbuf[m], sem_j[m])
            self.c = pltpu.make_async_copy(vals.at[g], vbuf[m], sem_k[m])

        def start(self):
            self.a.start()
            self.b.start()
            self.c.start()

        def wait(self):
            self.a.wait()
            self.b.wait()
            self.c.wait()

    def gat_copy(table, m):
        return pltpu.make_async_copy(table.at[sbuf[m]], rows.at[m],
                                     sem_g[m])

    def sct_copy(m):
        return pltpu.make_async_copy(rows.at[m], acc.at[dstb[m]], sem_s[m])

    def scale(m):
        for k in range(EBLK // 16):
            sl = pl.ds(k * 16, 16)
            dstb[m][sl] = dbuf[m][sl]
            vv = vbuf[m][sl]
            for q in range(16):
                e = k * 16 + q
                sv = jnp.broadcast_to(vv[q], (16,))
                rows[m, e, pl.ds(0, 16)] = rows[m, e, pl.ds(0, 16)] * sv
                rows[m, e, pl.ds(16, 16)] = rows[m, e, pl.ds(16, 16)] * sv

    def edge_phase(table):
        idx_copy(0, 0).start()
        idx_copy(1, 1).start()
        idx_copy(0, 0).wait()
        gat_copy(table, 0).start()

        idx_copy(1, 1).wait()
        gat_copy(table, 1).start()
        idx_copy(2, 2).start()
        gat_copy(table, 0).wait()
        scale(0)
        sct_copy(0).start(add=True)

        idx_copy(2, 2).wait()
        gat_copy(table, 2).start()
        idx_copy(3, 0).start()
        gat_copy(table, 1).wait()
        scale(1)
        sct_copy(1).start(add=True)

        def outer(t, _):
            jb = 2 + 3 * t
            for u in range(3):
                j = jb + u
                m = (2 + u) % 3
                m1 = u % 3
                m2 = (1 + u) % 3
                sct_copy(m1).wait()
                idx_copy(j + 1, m1).wait()
                gat_copy(table, m1).start()
                idx_copy(j + 2, m2).start()
                gat_copy(table, m).wait()
                scale(m)
                sct_copy(m).start(add=True)
            return 0
        lax.fori_loop(0, (NBLK - 3) // 3, outer, 0)

        sct_copy(0).wait()
        idx_copy(NBLK, 0).wait()
        gat_copy(table, 2).wait()
        scale(2)
        sct_copy(2).start(add=True)
        sct_copy(1).wait()
        sct_copy(2).wait()

    for l in range(N_LAYERS):
        table = ego0.at[c] if l == 0 else out.at[l - 1, c]
        edge_phase(table)
        plsc.subcore_barrier()

        pltpu.sync_copy(acc.at[pl.ds(s * ROWS_PER_TILE, ROWS_PER_TILE)],
                        out.at[l, c].at[pl.ds(s * ROWS_PER_TILE,
                                              ROWS_PER_TILE)])
        item_copy(acc, l + 1)
        if l < N_LAYERS - 1:
            zero_acc()
        plsc.subcore_barrier()

    def mean_rows(r0, nrows, ba, bb):
        pltpu.sync_copy(ego0.at[c].at[pl.ds(r0, nrows)], ba)

        for l in range(N_LAYERS):
            pltpu.sync_copy(out.at[l, c].at[pl.ds(r0, nrows)], bb)
            scalef = 0.25 if l == N_LAYERS - 1 else 1.0

            def add_row(i, _):
                for j in range(2):
                    a = ba[i, pl.ds(j * 16, 16)]
                    b = bb[i, pl.ds(j * 16, 16)]
                    ba[i, pl.ds(j * 16, 16)] = (a + b) * scalef
                return 0
            lax.fori_loop(0, nrows, add_row, 0)
        pltpu.sync_copy(ba, uout.at[pl.ds(r0, nrows), pl.ds(HALF * c, HALF)])

    def mean_chunk(i, _):
        mean_rows(s * UROWS + i * UCHUNK, UCHUNK, mba, mbb)
        return 0
    lax.fori_loop(0, 13, mean_chunk, 0)

    @pl.when(s < NS - 1)
    def _():
        mean_rows(s * UROWS + 13 * UCHUNK, UCHUNK, mba, mbb)

    @pl.when(s == NS - 1)
    def _():
        mean_rows(15 * UROWS + 13 * UCHUNK, LCHUNK,
                  mba.at[pl.ds(0, LCHUNK)], mbb.at[pl.ds(0, LCHUNK)])


@jax.jit
def _propagate(ego0, srcs, dsts, vals, zrows):
    f = pl.kernel(
        _body,
        out_type=(
            jax.ShapeDtypeStruct((N_LAYERS, NC, N_PAD, HALF), jnp.float32),
            jax.ShapeDtypeStruct((USER_NUM, EMB), jnp.float32),
            jax.ShapeDtypeStruct((N_LAYERS + 1, ITEM_NUM, EMB), jnp.float32),
        ),
        mesh=plsc.VectorSubcoreMesh(core_axis_name="c", subcore_axis_name="s",
                                    num_cores=NC, num_subcores=NS),
        compiler_params=pltpu.CompilerParams(use_tc_tiling_on_sc=False),
        scratch_types=[
            pltpu.VMEM_SHARED((N_PAD, HALF), jnp.float32),
            pltpu.VMEM((EBLK,), jnp.int32),
            pltpu.VMEM((EBLK,), jnp.int32),
            pltpu.VMEM((EBLK,), jnp.int32),
            pltpu.VMEM((EBLK,), jnp.int32),
            pltpu.VMEM((EBLK,), jnp.int32),
            pltpu.VMEM((EBLK,), jnp.int32),
            pltpu.VMEM((EBLK,), jnp.float32),
            pltpu.VMEM((EBLK,), jnp.float32),
            pltpu.VMEM((EBLK,), jnp.float32),
            pltpu.VMEM((EBLK,), jnp.int32),
            pltpu.VMEM((EBLK,), jnp.int32),
            pltpu.VMEM((EBLK,), jnp.int32),
            pltpu.VMEM((3, EBLK, HALF), jnp.float32),
            pltpu.VMEM((UCHUNK, HALF), jnp.float32),
            pltpu.VMEM((UCHUNK, HALF), jnp.float32),
        ] + [pltpu.SemaphoreType.DMA] * 15,
    )
    return f(ego0, srcs, dsts, vals, zrows)


def _blockify(x):
    xb = x.reshape(NS, NBLK, EBLK)
    return jnp.pad(xb, ((0, 0), (0, 1), (0, 0))).reshape(NS * NBLK_IO, EBLK)


def kernel(user_emb, item_emb, adj_values, adj_indices):
    zpad = jnp.zeros((N_PAD - N_NODES, HALF), jnp.float32)
    ego0 = jnp.stack([
        jnp.concatenate([user_emb[:, :HALF], item_emb[:, :HALF], zpad], axis=0),
        jnp.concatenate([user_emb[:, HALF:], item_emb[:, HALF:], zpad], axis=0),
    ])

    dst = adj_indices[0].astype(jnp.int32)
    src = adj_indices[1].astype(jnp.int32)
    val = adj_values.astype(jnp.float32)

    npad = E_PAD - N_EDGES
    spread = (jnp.arange(npad, dtype=jnp.int32) * 63) % N_NODES
    srcs = _blockify(jnp.concatenate([src, spread]))
    dsts = _blockify(jnp.concatenate([dst, spread]))
    vals = _blockify(jnp.concatenate([val, jnp.zeros((npad,), jnp.float32)]))

    zrows = jnp.zeros((ROWS_PER_TILE, HALF), jnp.float32)

    _, user_out, item_out = _propagate(ego0, srcs, dsts, vals, zrows)
    return (user_out, item_out)


# speedup vs baseline: 1.0006x; 1.0006x over previous
import functools

import jax
import jax.numpy as jnp
from jax import lax
from jax.experimental import pallas as pl
from jax.experimental.pallas import tpu as pltpu
from jax.experimental.pallas import tpu_sc as plsc

USER_NUM = 25000
ITEM_NUM = 25000
N_NODES = USER_NUM + ITEM_NUM
N_EDGES = 800000
EMB = 64
HALF = 32
N_LAYERS = 3

NC = 2
NS = 16
EBLK = 128
NBLK = 393
NBLK_IO = NBLK + 1
E_PAD = NS * NBLK * EBLK
N_PAD = 50176
ROWS_PER_TILE = N_PAD // NS
UROWS = 1568
ULAST = USER_NUM - 15 * UROWS
UCHUNK = 112
LCHUNK = USER_NUM - (15 * UROWS + 13 * UCHUNK)


def _body(ego0, srcs, dsts, vals, zrows, out, uout, iout,
          acc, sb0, sb1, sb2, db0, db1, db2, vb0, vb1, vb2,
          dstb0, dstb1, dstb2, rows, mba, mbb,
          si0, si1, si2, sj0, sj1, sj2, sk0, sk1, sk2,
          sg0, sg1, sg2, ss0, ss1, ss2):
    c = lax.axis_index("c")
    s = lax.axis_index("s")
    sbuf = (sb0, sb1, sb2)
    dbuf = (db0, db1, db2)
    vbuf = (vb0, vb1, vb2)
    dstb = (dstb0, dstb1, dstb2)
    sem_i = (si0, si1, si2)
    sem_j = (sj0, sj1, sj2)
    sem_k = (sk0, sk1, sk2)
    sem_g = (sg0, sg1, sg2)
    sem_s = (ss0, ss1, ss2)

    def zero_acc():
        pltpu.sync_copy(zrows, acc.at[pl.ds(s * ROWS_PER_TILE,
                                            ROWS_PER_TILE)])

    def item_copy(src_table, l):
        @pl.when(s < NS - 1)
        def _():
            pltpu.sync_copy(
                src_table.at[pl.ds(USER_NUM + s * UROWS, UROWS)],
                iout.at[l].at[pl.ds(s * UROWS, UROWS),
                              pl.ds(HALF * c, HALF)])

        @pl.when(s == NS - 1)
        def _():
            pltpu.sync_copy(
                src_table.at[pl.ds(USER_NUM + 15 * UROWS, ULAST)],
                iout.at[l].at[pl.ds(15 * UROWS, ULAST),
                              pl.ds(HALF * c, HALF)])

    zero_acc()
    item_copy(ego0.at[c], 0)
    plsc.subcore_barrier()

    class idx_copy:
        def __init__(self, j, m):
            g = s * NBLK_IO + j
            self.a = pltpu.make_async_copy(srcs.at[g], sbuf[m], sem_i[m])
            self.b = pltpu.make_async_copy(dsts.at[g], dbuf[m], sem_j[m])
            self.c = pltpu.make_async_copy(vals.at[g], vbuf[m], sem_k[m])

        def start(self):
            self.a.start()
            self.b.start()
            self.c.start()

        def wait(self):
            self.a.wait()
            self.b.wait()
            self.c.wait()

    def gat_copy(table, m):
        return pltpu.make_async_copy(table.at[sbuf[m]], rows.at[m],
                                     sem_g[m])

    def sct_copy(m):
        return pltpu.make_async_copy(rows.at[m], acc.at[dstb[m]], sem_s[m])

    def scale(m):
        for k in range(EBLK // 16):
            sl = pl.ds(k * 16, 16)
            dstb[m][sl] = dbuf[m][sl]
            vv = vbuf[m][sl]
            for q in range(16):
                e = k * 16 + q
                sv = jnp.broadcast_to(vv[q], (16,))
                rows[m, e, pl.ds(0, 16)] = rows[m, e, pl.ds(0, 16)] * sv
                rows[m, e, pl.ds(16, 16)] = rows[m, e, pl.ds(16, 16)] * sv

    def edge_phase(table):
        idx_copy(0, 0).start()
        idx_copy(1, 1).start()
        idx_copy(0, 0).wait()
        gat_copy(table, 0).start()

        idx_copy(1, 1).wait()
        gat_copy(table, 1).start()
        idx_copy(2, 2).start()
        gat_copy(table, 0).wait()
        scale(0)
        sct_copy(0).start(add=True)

        idx_copy(2, 2).wait()
        gat_copy(table, 2).start()
        idx_copy(3, 0).start()
        gat_copy(table, 1).wait()
        scale(1)
        sct_copy(1).start(add=True)

        def outer(t, _):
            jb = 2 + 3 * t
            for u in range(3):
                j = jb + u
                m = (2 + u) % 3
                m1 = u % 3
                m2 = (1 + u) % 3
                sct_copy(m1).wait()
                idx_copy(j + 1, m1).wait()
                gat_copy(table, m1).start()
                idx_copy(j + 2, m2).start()
                gat_copy(table, m).wait()
                scale(m)
                sct_copy(m).start(add=True)
            return 0
        lax.fori_loop(0, (NBLK - 3) // 3, outer, 0)

        sct_copy(0).wait()
        idx_copy(NBLK, 0).wait()
        gat_copy(table, 2).wait()
        scale(2)
        sct_copy(2).start(add=True)
        sct_copy(1).wait()
        sct_copy(2).wait()

    for l in range(N_LAYERS):
        table = ego0.at[c] if l == 0 else out.at[l - 1, c]
        edge_phase(table)
        plsc.subcore_barrier()

        if l < N_LAYERS - 1:
            pltpu.sync_copy(acc.at[pl.ds(s * ROWS_PER_TILE, ROWS_PER_TILE)],
                            out.at[l, c].at[pl.ds(s * ROWS_PER_TILE,
                                                  ROWS_PER_TILE)])
        else:
            @pl.when(s * ROWS_PER_TILE < NS * UROWS)
            def _():
                pltpu.sync_copy(
                    acc.at[pl.ds(s * ROWS_PER_TILE, ROWS_PER_TILE)],
                    out.at[l, c].at[pl.ds(s * ROWS_PER_TILE,
                                          ROWS_PER_TILE)])
        item_copy(acc, l + 1)
        if l < N_LAYERS - 1:
            zero_acc()
        plsc.subcore_barrier()

    def mean_rows(r0, nrows, ba, bb):
        pltpu.sync_copy(ego0.at[c].at[pl.ds(r0, nrows)], ba)

        for l in range(N_LAYERS):
            pltpu.sync_copy(out.at[l, c].at[pl.ds(r0, nrows)], bb)
            scalef = 0.25 if l == N_LAYERS - 1 else 1.0

            def add_row(i, _):
                for j in range(2):
                    a = ba[i, pl.ds(j * 16, 16)]
                    b = bb[i, pl.ds(j * 16, 16)]
                    ba[i, pl.ds(j * 16, 16)] = (a + b) * scalef
                return 0
            lax.fori_loop(0, nrows, add_row, 0)
        pltpu.sync_copy(ba, uout.at[pl.ds(r0, nrows), pl.ds(HALF * c, HALF)])

    def mean_chunk(i, _):
        mean_rows(s * UROWS + i * UCHUNK, UCHUNK, mba, mbb)
        return 0
    lax.fori_loop(0, 13, mean_chunk, 0)

    @pl.when(s < NS - 1)
    def _():
        mean_rows(s * UROWS + 13 * UCHUNK, UCHUNK, mba, mbb)

    @pl.when(s == NS - 1)
    def _():
        mean_rows(15 * UROWS + 13 * UCHUNK, LCHUNK,
                  mba.at[pl.ds(0, LCHUNK)], mbb.at[pl.ds(0, LCHUNK)])


@jax.jit
def _propagate(ego0, srcs, dsts, vals, zrows):
    f = pl.kernel(
        _body,
        out_type=(
            jax.ShapeDtypeStruct((N_LAYERS, NC, N_PAD, HALF), jnp.float32),
            jax.ShapeDtypeStruct((USER_NUM, EMB), jnp.float32),
            jax.ShapeDtypeStruct((N_LAYERS + 1, ITEM_NUM, EMB), jnp.float32),
        ),
        mesh=plsc.VectorSubcoreMesh(core_axis_name="c", subcore_axis_name="s",
                                    num_cores=NC, num_subcores=NS),
        compiler_params=pltpu.CompilerParams(use_tc_tiling_on_sc=False),
        scratch_types=[
            pltpu.VMEM_SHARED((N_PAD, HALF), jnp.float32),
            pltpu.VMEM((EBLK,), jnp.int32),
            pltpu.VMEM((EBLK,), jnp.int32),
            pltpu.VMEM((EBLK,), jnp.int32),
            pltpu.VMEM((EBLK,), jnp.int32),
            pltpu.VMEM((EBLK,), jnp.int32),
            pltpu.VMEM((EBLK,), jnp.int32),
            pltpu.VMEM((EBLK,), jnp.float32),
            pltpu.VMEM((EBLK,), jnp.float32),
            pltpu.VMEM((EBLK,), jnp.float32),
            pltpu.VMEM((EBLK,), jnp.int32),
            pltpu.VMEM((EBLK,), jnp.int32),
            pltpu.VMEM((EBLK,), jnp.int32),
            pltpu.VMEM((3, EBLK, HALF), jnp.float32),
            pltpu.VMEM((UCHUNK, HALF), jnp.float32),
            pltpu.VMEM((UCHUNK, HALF), jnp.float32),
        ] + [pltpu.SemaphoreType.DMA] * 15,
    )
    return f(ego0, srcs, dsts, vals, zrows)


def _blockify(x):
    xb = x.reshape(NS, NBLK, EBLK)
    return jnp.pad(xb, ((0, 0), (0, 1), (0, 0))).reshape(NS * NBLK_IO, EBLK)


def kernel(user_emb, item_emb, adj_values, adj_indices):
    zpad = jnp.zeros((N_PAD - N_NODES, HALF), jnp.float32)
    ego0 = jnp.stack([
        jnp.concatenate([user_emb[:, :HALF], item_emb[:, :HALF], zpad], axis=0),
        jnp.concatenate([user_emb[:, HALF:], item_emb[:, HALF:], zpad], axis=0),
    ])

    dst = adj_indices[0].astype(jnp.int32)
    src = adj_indices[1].astype(jnp.int32)
    val = adj_values.astype(jnp.float32)

    npad = E_PAD - N_EDGES
    spread = (jnp.arange(npad, dtype=jnp.int32) * 63) % N_NODES
    srcs = _blockify(jnp.concatenate([src, spread]))
    dsts = _blockify(jnp.concatenate([dst, spread]))
    vals = _blockify(jnp.concatenate([val, jnp.zeros((npad,), jnp.float32)]))

    zrows = jnp.zeros((ROWS_PER_TILE, HALF), jnp.float32)

    _, user_out, item_out = _propagate(ego0, srcs, dsts, vals, zrows)
    return (user_out, item_out)
